# baseline (speedup 1.0000x reference)
import functools

import jax
import jax.numpy as jnp
from jax import lax
from jax.experimental import pallas as pl
from jax.experimental.pallas import tpu as pltpu
from jax.experimental.pallas import tpu_sc as plsc

N = 100000
E = 1600000
FIN = 16
HID = 32
HALF = 16
FOUT = 2

NCORES = 2
NSUB = 16

NP = 100096
NPF = NP // 8

DEG_SLICE = NP // NSUB
EDGES_PER_W = E // (NCORES * NSUB)
DEG_CHUNK = 5000
DEG_NCHUNK = EDGES_PER_W // DEG_CHUNK
DEG_BUF = 6256

DINV_SLICE = NP // (NCORES * NSUB)
DINV_BUF = 3136

EDGES_PER_SUB = E // NSUB
AGG_CHUNK = 800
AGG_NCHUNK = EDGES_PER_SUB // AGG_CHUNK
ROWS_PER_SUB = NP // NSUB
_STAGES = [(j * 800, 800) for j in range(7)] + [(5600, 656)]

_MESH = dict(
    mesh=plsc.VectorSubcoreMesh(core_axis_name="c", subcore_axis_name="s"),
    compiler_params=pltpu.CompilerParams(
        use_tc_tiling_on_sc=False, needs_layout_passes=False
    ),
)


def _fill1d(ref, n, value):
    vec = jnp.full((16,), value, dtype=ref.dtype)

    def body(i, c):
        ref[pl.ds(i * 16, 16)] = vec
        return c

    lax.fori_loop(0, n // 16, body, 0)


def _zero_rows(ref, nrows):
    z = jnp.zeros((16,), dtype=ref.dtype)

    def body(i, c):
        ref[i, :] = z
        return c

    lax.fori_loop(0, nrows, body, 0)


@functools.partial(
    pl.kernel,
    out_type=jax.ShapeDtypeStruct((NCORES * NP,), jnp.float32),
    scratch_types=[
        pltpu.VMEM((DEG_CHUNK,), jnp.int32),
        pltpu.VMEM((DEG_BUF,), jnp.float32),
        pltpu.VMEM_SHARED((NP,), jnp.float32),
    ],
    **_MESH,
)
def _deg_kernel(dst_hbm, out, idx_v, val_v, acc):
    cid = lax.axis_index("c")
    sid = lax.axis_index("s")
    base_r = sid * DEG_SLICE
    _fill1d(val_v, DEG_BUF, 0.0)
    pltpu.sync_copy(val_v, acc.at[pl.ds(base_r, DEG_SLICE)])
    _fill1d(val_v, DEG_BUF, 1.0)
    plsc.subcore_barrier()

    def chunk(k, c):
        base = (cid * NSUB + sid) * EDGES_PER_W + k * DEG_CHUNK
        pltpu.sync_copy(dst_hbm.at[pl.ds(base, DEG_CHUNK)], idx_v)
        pltpu.sync_copy(val_v.at[pl.ds(0, DEG_CHUNK)], acc.at[idx_v], add=True)
        return c

    lax.fori_loop(0, DEG_NCHUNK, chunk, 0)
    plsc.subcore_barrier()
    pltpu.sync_copy(acc.at[pl.ds(base_r, DEG_SLICE)], val_v)
    pltpu.sync_copy(val_v, out.at[pl.ds(cid * NP + base_r, DEG_SLICE)])


def _rsqrt16(d):
    i = plsc.bitcast(d, jnp.int32)
    i = jnp.int32(0x5F3759DF) - (i >> 1)
    y = plsc.bitcast(i, jnp.float32)
    for _ in range(3):
        y = y * (1.5 - 0.5 * d * y * y)
    return y


@functools.partial(
    pl.kernel,
    out_type=jax.ShapeDtypeStruct((NP, HALF), jnp.float32),
    scratch_types=[
        pltpu.VMEM((DINV_BUF,), jnp.float32),
        pltpu.VMEM((DINV_BUF,), jnp.float32),
        pltpu.VMEM((16,), jnp.float32),
        pltpu.VMEM((DINV_BUF, HALF), jnp.float32),
        pltpu.SemaphoreType.DMA,
    ],
    **_MESH,
)
def _dinv_kernel(degp, out, p0b, p1b, ybuf, ob, sem):
    cid = lax.axis_index("c")
    sid = lax.axis_index("s")
    t = cid * NSUB + sid
    base = t * DINV_SLICE
    pltpu.async_copy(degp.at[pl.ds(base, DINV_SLICE)], p0b.at[pl.ds(0, DINV_SLICE)], sem)
    pltpu.async_copy(degp.at[pl.ds(NP + base, DINV_SLICE)], p1b.at[pl.ds(0, DINV_SLICE)], sem)
    pltpu.make_async_copy(degp.at[pl.ds(0, DINV_SLICE)], p0b.at[pl.ds(0, DINV_SLICE)], sem).wait()
    pltpu.make_async_copy(degp.at[pl.ds(0, DINV_SLICE)], p1b.at[pl.ds(0, DINV_SLICE)], sem).wait()

    def group(g, c):
        d = p0b[pl.ds(g * 16, 16)] + p1b[pl.ds(g * 16, 16)] + 1.0
        yv = _rsqrt16(d)
        for l in range(16):
            ob[g * 16 + l, :] = jnp.full((16,), yv[l], jnp.float32)
        return c

    lax.fori_loop(0, DINV_BUF // 16, group, 0)
    pltpu.sync_copy(ob.at[pl.ds(0, DINV_SLICE)], out.at[pl.ds(base, DINV_SLICE)])


@functools.partial(
    pl.kernel,
    out_type=(
        jax.ShapeDtypeStruct((NP, HALF), jnp.float32),
        jax.ShapeDtypeStruct((NP, HALF), jnp.float32),
    ),
    scratch_types=[
        pltpu.VMEM((AGG_CHUNK,), jnp.int32),
        pltpu.VMEM((AGG_CHUNK,), jnp.int32),
        pltpu.VMEM((AGG_CHUNK,), jnp.int32),
        pltpu.VMEM((AGG_CHUNK,), jnp.int32),
        pltpu.VMEM((AGG_CHUNK, HALF), jnp.float32),
        pltpu.VMEM((AGG_CHUNK, HALF), jnp.float32),
        pltpu.VMEM_SHARED((NP, HALF), jnp.float32),
        pltpu.SemaphoreType.DMA,
        pltpu.SemaphoreType.DMA,
        pltpu.SemaphoreType.DMA,
        pltpu.SemaphoreType.DMA,
        pltpu.SemaphoreType.DMA,
        pltpu.SemaphoreType.DMA,
    ],
    **_MESH,
)
def _agg_kernel(
    h0, h1, src_hbm, dst_hbm, z0, z1,
    src_b0, src_b1, dst_b0, dst_b1, rows0, rows1,
    acc, si0, si1, sg0, sg1, ss0, ss1,
):
    cid = lax.axis_index("c")
    sid = lax.axis_index("s")
    r0 = sid * ROWS_PER_SUB
    rows = (rows0, rows1)
    ss = (ss0, ss1)
    slots = (
        (src_b0, dst_b0, rows0, si0, sg0, ss0),
        (src_b1, dst_b1, rows1, si1, sg1, ss1),
    )

    _zero_rows(rows0, AGG_CHUNK)
    for off, size in _STAGES:
        pltpu.async_copy(rows0.at[pl.ds(0, size)], acc.at[pl.ds(r0 + off, size)], ss0)
    for off, size in _STAGES:
        pltpu.make_async_copy(rows0.at[pl.ds(0, size)], acc.at[pl.ds(r0 + off, size)], ss0).wait()
    plsc.subcore_barrier()

    def run(h_hbm):
        def chunk_ops(k, b, wait_prev):
            sb, db, rb, si, sg, so = slots[b]
            base = sid * EDGES_PER_SUB + k * AGG_CHUNK
            if wait_prev:
                pltpu.make_async_copy(rb, acc.at[db], so).wait()
            pltpu.async_copy(src_hbm.at[pl.ds(base, AGG_CHUNK)], sb, si)
            pltpu.async_copy(dst_hbm.at[pl.ds(base, AGG_CHUNK)], db, si)
            pltpu.make_async_copy(src_hbm.at[pl.ds(0, AGG_CHUNK)], sb, si).wait()
            pltpu.make_async_copy(dst_hbm.at[pl.ds(0, AGG_CHUNK)], db, si).wait()
            pltpu.async_copy(h_hbm.at[sb], rb, sg)
            pltpu.make_async_copy(h_hbm.at[sb], rb, sg).wait()
            pltpu.async_copy(rb, acc.at[db], so, add=True)

        chunk_ops(0, 0, False)
        chunk_ops(1, 1, False)

        def pair(kk, c):
            chunk_ops(2 * kk, 0, True)
            chunk_ops(2 * kk + 1, 1, True)
            return c

        lax.fori_loop(1, AGG_NCHUNK // 2, pair, 0)
        chunk_ops(AGG_NCHUNK - 1, 0, True)
        pltpu.make_async_copy(rows1, acc.at[dst_b1], ss1).wait()
        pltpu.make_async_copy(rows0, acc.at[dst_b0], ss0).wait()

    @pl.when(cid == 0)
    def _():
        run(h0)

    @pl.when(cid == 1)
    def _():
        run(h1)

    plsc.subcore_barrier()

    def copy_out(z_hbm):
        for j, (off, size) in enumerate(_STAGES):
            rb = rows[j % 2]
            so = ss[j % 2]
            if j >= 2:
                psize = _STAGES[j - 2][1]
                pltpu.make_async_copy(rb.at[pl.ds(0, psize)], z_hbm.at[pl.ds(0, psize)], so).wait()
            pltpu.sync_copy(acc.at[pl.ds(r0 + off, size)], rb.at[pl.ds(0, size)])
            pltpu.async_copy(rb.at[pl.ds(0, size)], z_hbm.at[pl.ds(r0 + off, size)], so)
        for j in (len(_STAGES) - 2, len(_STAGES) - 1):
            rb = rows[j % 2]
            so = ss[j % 2]
            size = _STAGES[j][1]
            pltpu.make_async_copy(rb.at[pl.ds(0, size)], z_hbm.at[pl.ds(0, size)], so).wait()

    @pl.when(cid == 0)
    def _():
        copy_out(z0)

    @pl.when(cid == 1)
    def _():
        copy_out(z1)


BLK = 736
GRID = NPF // BLK


def _nspec():
    return pl.BlockSpec((BLK, 128), lambda i: (i, 0))


def _wspec(shape):
    return pl.BlockSpec(shape, lambda i: (0, 0))


def _tc1_call(xf, dvf, a0, a1):
    def body(x_ref, dv_ref, a0_ref, a1_ref, h0_ref, h1_ref):
        xv = x_ref[...]
        dv = dv_ref[...]
        h0_ref[...] = jnp.dot(xv, a0_ref[...], preferred_element_type=jnp.float32) * dv
        h1_ref[...] = jnp.dot(xv, a1_ref[...], preferred_element_type=jnp.float32) * dv

    return pl.pallas_call(
        body,
        grid=(GRID,),
        in_specs=[_nspec(), _nspec(), _wspec((128, 128)), _wspec((128, 128))],
        out_specs=[_nspec(), _nspec()],
        out_shape=[
            jax.ShapeDtypeStruct((NPF, 128), jnp.float32),
            jax.ShapeDtypeStruct((NPF, 128), jnp.float32),
        ],
    )(xf, dvf, a0, a1)


def _tc2_call(z0f, z1f, h0f, h1f, dvf, b0, b1, w00, w01, w10, w11):
    def body(z0_ref, z1_ref, h0_ref, h1_ref, dv_ref, b0_ref, b1_ref,
             w00_ref, w01_ref, w10_ref, w11_ref, g0_ref, g1_ref):
        dv = dv_ref[...]
        a0 = jax.nn.sigmoid((z0_ref[...] + h0_ref[...]) * dv + b0_ref[...])
        a1 = jax.nn.sigmoid((z1_ref[...] + h1_ref[...]) * dv + b1_ref[...])
        g0_ref[...] = (
            jnp.dot(a0, w00_ref[...], preferred_element_type=jnp.float32)
            + jnp.dot(a1, w10_ref[...], preferred_element_type=jnp.float32)
        ) * dv
        g1_ref[...] = (
            jnp.dot(a0, w01_ref[...], preferred_element_type=jnp.float32)
            + jnp.dot(a1, w11_ref[...], preferred_element_type=jnp.float32)
        ) * dv

    return pl.pallas_call(
        body,
        grid=(GRID,),
        in_specs=[
            _nspec(), _nspec(), _nspec(), _nspec(), _nspec(),
            _wspec((1, 128)), _wspec((1, 128)),
            _wspec((128, 128)), _wspec((128, 128)),
            _wspec((128, 128)), _wspec((128, 128)),
        ],
        out_specs=[_nspec(), _nspec()],
        out_shape=[
            jax.ShapeDtypeStruct((NPF, 128), jnp.float32),
            jax.ShapeDtypeStruct((NPF, 128), jnp.float32),
        ],
    )(z0f, z1f, h0f, h1f, dvf, b0, b1, w00, w01, w10, w11)


def _tc3_call(y0f, y1f, g0f, g1f, dvf, b0, b1, l0, l1, blx):
    def body(y0_ref, y1_ref, g0_ref, g1_ref, dv_ref, b0_ref, b1_ref,
             l0_ref, l1_ref, bl_ref, o_ref):
        dv = dv_ref[...]
        a0 = jax.nn.sigmoid((y0_ref[...] + g0_ref[...]) * dv + b0_ref[...])
        a1 = jax.nn.sigmoid((y1_ref[...] + g1_ref[...]) * dv + b1_ref[...])
        o_ref[...] = jax.nn.sigmoid(
            jnp.dot(a0, l0_ref[...], preferred_element_type=jnp.float32)
            + jnp.dot(a1, l1_ref[...], preferred_element_type=jnp.float32)
            + bl_ref[...]
        )

    return pl.pallas_call(
        body,
        grid=(GRID,),
        in_specs=[
            _nspec(), _nspec(), _nspec(), _nspec(), _nspec(),
            _wspec((1, 128)), _wspec((1, 128)),
            _wspec((128, 2 * 8)), _wspec((128, 2 * 8)), _wspec((1, 2 * 8)),
        ],
        out_specs=pl.BlockSpec((BLK, 2 * 8), lambda i: (i, 0)),
        out_shape=jax.ShapeDtypeStruct((NPF, 2 * 8), jnp.float32),
    )(y0f, y1f, g0f, g1f, dvf, b0, b1, l0, l1, blx)


def kernel(x, edge_index, W1, b1, W2, b2, Wl, bl):
    f32 = jnp.float32
    ei = edge_index.astype(jnp.int32)
    src = ei[0]
    dst = ei[1]

    degp = _deg_kernel(dst)
    dinvx = _dinv_kernel(degp)
    dvf = dinvx.reshape(NPF, 128)

    xf = jnp.pad(x, ((0, NP - N), (0, 0))).reshape(NPF, 128)
    eye8 = jnp.eye(8, dtype=f32)
    a0 = jnp.kron(eye8, W1[:, :HALF])
    a1 = jnp.kron(eye8, W1[:, HALF:])
    h0f, h1f = _tc1_call(xf, dvf, a0, a1)

    z0, z1 = _agg_kernel(
        h0f.reshape(NP, HALF), h1f.reshape(NP, HALF), src, dst
    )

    b1x0 = jnp.tile(b1[:HALF], 8).reshape(1, 128)
    b1x1 = jnp.tile(b1[HALF:], 8).reshape(1, 128)
    w00 = jnp.kron(eye8, W2[:HALF, :HALF])
    w01 = jnp.kron(eye8, W2[:HALF, HALF:])
    w10 = jnp.kron(eye8, W2[HALF:, :HALF])
    w11 = jnp.kron(eye8, W2[HALF:, HALF:])
    g0f, g1f = _tc2_call(
        z0.reshape(NPF, 128), z1.reshape(NPF, 128), h0f, h1f, dvf,
        b1x0, b1x1, w00, w01, w10, w11,
    )

    y0, y1 = _agg_kernel(
        g0f.reshape(NP, HALF), g1f.reshape(NP, HALF), src, dst
    )

    b2x0 = jnp.tile(b2[:HALF], 8).reshape(1, 128)
    b2x1 = jnp.tile(b2[HALF:], 8).reshape(1, 128)
    l0 = jnp.kron(eye8, Wl[:HALF, :])
    l1 = jnp.kron(eye8, Wl[HALF:, :])
    blx = jnp.tile(bl, 8).reshape(1, 16)
    outf = _tc3_call(
        y0.reshape(NPF, 128), y1.reshape(NPF, 128), g0f, g1f, dvf,
        b2x0, b2x1, l0, l1, blx,
    )
    return outf.reshape(NP, FOUT)[:N]


# speedup vs baseline: 1.0524x; 1.0524x over previous
import functools

import jax
import jax.numpy as jnp
from jax import lax
from jax.experimental import pallas as pl
from jax.experimental.pallas import tpu as pltpu
from jax.experimental.pallas import tpu_sc as plsc

N = 100000
E = 1600000
FIN = 16
HID = 32
HALF = 16
FOUT = 2

NCORES = 2
NSUB = 16

NP = 100096
NPF = NP // 8

DEG_SLICE = NP // NSUB
EDGES_PER_W = E // (NCORES * NSUB)
DEG_CHUNK = 5000
DEG_NCHUNK = EDGES_PER_W // DEG_CHUNK
DEG_BUF = 6256

DINV_SLICE = NP // (NCORES * NSUB)
DINV_BUF = 3136

EDGES_PER_SUB = E // NSUB
AGG_CHUNK = 800
AGG_NCHUNK = EDGES_PER_SUB // AGG_CHUNK
ROWS_PER_SUB = NP // NSUB
_STAGES = [(j * 800, 800) for j in range(7)] + [(5600, 656)]

_MESH = dict(
    mesh=plsc.VectorSubcoreMesh(core_axis_name="c", subcore_axis_name="s"),
    compiler_params=pltpu.CompilerParams(
        use_tc_tiling_on_sc=False, needs_layout_passes=False
    ),
)


def _fill1d(ref, n, value):
    vec = jnp.full((16,), value, dtype=ref.dtype)

    def body(i, c):
        ref[pl.ds(i * 16, 16)] = vec
        return c

    lax.fori_loop(0, n // 16, body, 0)


def _zero_rows(ref, nrows):
    z = jnp.zeros((16,), dtype=ref.dtype)

    def body(i, c):
        ref[i, :] = z
        return c

    lax.fori_loop(0, nrows, body, 0)


@functools.partial(
    pl.kernel,
    out_type=jax.ShapeDtypeStruct((NCORES * NP,), jnp.float32),
    scratch_types=[
        pltpu.VMEM((DEG_CHUNK,), jnp.int32),
        pltpu.VMEM((DEG_BUF,), jnp.float32),
        pltpu.VMEM_SHARED((NP,), jnp.float32),
    ],
    **_MESH,
)
def _deg_kernel(dst_hbm, out, idx_v, val_v, acc):
    cid = lax.axis_index("c")
    sid = lax.axis_index("s")
    base_r = sid * DEG_SLICE
    _fill1d(val_v, DEG_BUF, 0.0)
    pltpu.sync_copy(val_v, acc.at[pl.ds(base_r, DEG_SLICE)])
    _fill1d(val_v, DEG_BUF, 1.0)
    plsc.subcore_barrier()

    def chunk(k, c):
        base = E + (cid * NSUB + sid) * EDGES_PER_W + k * DEG_CHUNK
        pltpu.sync_copy(dst_hbm.at[pl.ds(base, DEG_CHUNK)], idx_v)
        pltpu.sync_copy(val_v.at[pl.ds(0, DEG_CHUNK)], acc.at[idx_v], add=True)
        return c

    lax.fori_loop(0, DEG_NCHUNK, chunk, 0)
    plsc.subcore_barrier()
    pltpu.sync_copy(acc.at[pl.ds(base_r, DEG_SLICE)], val_v)
    pltpu.sync_copy(val_v, out.at[pl.ds(cid * NP + base_r, DEG_SLICE)])


def _rsqrt16(d):
    i = plsc.bitcast(d, jnp.int32)
    i = jnp.int32(0x5F3759DF) - (i >> 1)
    y = plsc.bitcast(i, jnp.float32)
    for _ in range(3):
        y = y * (1.5 - 0.5 * d * y * y)
    return y


DINV_ROWS = DINV_SLICE * HALF // 128


@functools.partial(
    pl.kernel,
    out_type=jax.ShapeDtypeStruct((NPF, 128), jnp.float32),
    scratch_types=[
        pltpu.VMEM((DINV_BUF,), jnp.float32),
        pltpu.VMEM((DINV_BUF,), jnp.float32),
        pltpu.VMEM((DINV_BUF // 8, 128), jnp.float32),
        pltpu.SemaphoreType.DMA,
    ],
    **_MESH,
)
def _dinv_kernel(degp, out, p0b, p1b, ob, sem):
    cid = lax.axis_index("c")
    sid = lax.axis_index("s")
    t = cid * NSUB + sid
    base = t * DINV_SLICE
    pltpu.async_copy(degp.at[pl.ds(base, DINV_SLICE)], p0b.at[pl.ds(0, DINV_SLICE)], sem)
    pltpu.async_copy(degp.at[pl.ds(NP + base, DINV_SLICE)], p1b.at[pl.ds(0, DINV_SLICE)], sem)
    pltpu.make_async_copy(degp.at[pl.ds(0, DINV_SLICE)], p0b.at[pl.ds(0, DINV_SLICE)], sem).wait()
    pltpu.make_async_copy(degp.at[pl.ds(0, DINV_SLICE)], p1b.at[pl.ds(0, DINV_SLICE)], sem).wait()

    def group(g, c):
        d = p0b[pl.ds(g * 16, 16)] + p1b[pl.ds(g * 16, 16)] + 1.0
        yv = _rsqrt16(d)
        for l in range(16):
            ob[2 * g + l // 8, pl.ds((l % 8) * 16, 16)] = jnp.full(
                (16,), yv[l], jnp.float32
            )
        return c

    lax.fori_loop(0, DINV_BUF // 16, group, 0)
    pltpu.sync_copy(ob.at[pl.ds(0, DINV_ROWS)], out.at[pl.ds(t * DINV_ROWS, DINV_ROWS)])


@functools.partial(
    pl.kernel,
    out_type=(
        jax.ShapeDtypeStruct((NP, HALF), jnp.float32),
        jax.ShapeDtypeStruct((NP, HALF), jnp.float32),
    ),
    scratch_types=[
        pltpu.VMEM((AGG_CHUNK,), jnp.int32),
        pltpu.VMEM((AGG_CHUNK,), jnp.int32),
        pltpu.VMEM((AGG_CHUNK,), jnp.int32),
        pltpu.VMEM((AGG_CHUNK,), jnp.int32),
        pltpu.VMEM((AGG_CHUNK, HALF), jnp.float32),
        pltpu.VMEM((AGG_CHUNK, HALF), jnp.float32),
        pltpu.VMEM_SHARED((NP, HALF), jnp.float32),
        pltpu.SemaphoreType.DMA,
        pltpu.SemaphoreType.DMA,
        pltpu.SemaphoreType.DMA,
        pltpu.SemaphoreType.DMA,
        pltpu.SemaphoreType.DMA,
        pltpu.SemaphoreType.DMA,
    ],
    **_MESH,
)
def _agg_kernel(
    h0, h1, src_hbm, dst_hbm, z0, z1,
    src_b0, src_b1, dst_b0, dst_b1, rows0, rows1,
    acc, si0, si1, sg0, sg1, ss0, ss1,
):
    cid = lax.axis_index("c")
    sid = lax.axis_index("s")
    r0 = sid * ROWS_PER_SUB
    rows = (rows0, rows1)
    ss = (ss0, ss1)
    slots = (
        (src_b0, dst_b0, rows0, si0, sg0, ss0),
        (src_b1, dst_b1, rows1, si1, sg1, ss1),
    )

    _zero_rows(rows0, AGG_CHUNK)
    for off, size in _STAGES:
        pltpu.async_copy(rows0.at[pl.ds(0, size)], acc.at[pl.ds(r0 + off, size)], ss0)
    for off, size in _STAGES:
        pltpu.make_async_copy(rows0.at[pl.ds(0, size)], acc.at[pl.ds(r0 + off, size)], ss0).wait()
    plsc.subcore_barrier()

    def run(h_hbm):
        def chunk_ops(k, b, wait_prev):
            sb, db, rb, si, sg, so = slots[b]
            base = sid * EDGES_PER_SUB + k * AGG_CHUNK
            if wait_prev:
                pltpu.make_async_copy(rb, acc.at[db], so).wait()
            pltpu.async_copy(src_hbm.at[pl.ds(base, AGG_CHUNK)], sb, si)
            pltpu.async_copy(dst_hbm.at[pl.ds(E + base, AGG_CHUNK)], db, si)
            pltpu.make_async_copy(src_hbm.at[pl.ds(0, AGG_CHUNK)], sb, si).wait()
            pltpu.make_async_copy(dst_hbm.at[pl.ds(0, AGG_CHUNK)], db, si).wait()
            pltpu.async_copy(h_hbm.at[sb], rb, sg)
            pltpu.make_async_copy(h_hbm.at[sb], rb, sg).wait()
            pltpu.async_copy(rb, acc.at[db], so, add=True)

        chunk_ops(0, 0, False)
        chunk_ops(1, 1, False)

        def pair(kk, c):
            chunk_ops(2 * kk, 0, True)
            chunk_ops(2 * kk + 1, 1, True)
            return c

        lax.fori_loop(1, AGG_NCHUNK // 2, pair, 0)
        chunk_ops(AGG_NCHUNK - 1, 0, True)
        pltpu.make_async_copy(rows1, acc.at[dst_b1], ss1).wait()
        pltpu.make_async_copy(rows0, acc.at[dst_b0], ss0).wait()

    @pl.when(cid == 0)
    def _():
        run(h0)

    @pl.when(cid == 1)
    def _():
        run(h1)

    plsc.subcore_barrier()

    def copy_out(z_hbm):
        for j, (off, size) in enumerate(_STAGES):
            rb = rows[j % 2]
            so = ss[j % 2]
            if j >= 2:
                psize = _STAGES[j - 2][1]
                pltpu.make_async_copy(rb.at[pl.ds(0, psize)], z_hbm.at[pl.ds(0, psize)], so).wait()
            pltpu.sync_copy(acc.at[pl.ds(r0 + off, size)], rb.at[pl.ds(0, size)])
            pltpu.async_copy(rb.at[pl.ds(0, size)], z_hbm.at[pl.ds(r0 + off, size)], so)
        for j in (len(_STAGES) - 2, len(_STAGES) - 1):
            rb = rows[j % 2]
            so = ss[j % 2]
            size = _STAGES[j][1]
            pltpu.make_async_copy(rb.at[pl.ds(0, size)], z_hbm.at[pl.ds(0, size)], so).wait()

    @pl.when(cid == 0)
    def _():
        copy_out(z0)

    @pl.when(cid == 1)
    def _():
        copy_out(z1)


BLK = 736
GRID = NPF // BLK


def _nspec():
    return pl.BlockSpec((BLK, 128), lambda i: (i, 0))


def _wspec(shape):
    return pl.BlockSpec(shape, lambda i: (0, 0))


def _tc1_call(xf, dvf, a0, a1):
    def body(x_ref, dv_ref, a0_ref, a1_ref, h0_ref, h1_ref):
        xv = x_ref[...]
        dv = dv_ref[...]
        h0_ref[...] = jnp.dot(xv, a0_ref[...], preferred_element_type=jnp.float32) * dv
        h1_ref[...] = jnp.dot(xv, a1_ref[...], preferred_element_type=jnp.float32) * dv

    return pl.pallas_call(
        body,
        grid=(GRID,),
        in_specs=[_nspec(), _nspec(), _wspec((128, 128)), _wspec((128, 128))],
        out_specs=[_nspec(), _nspec()],
        out_shape=[
            jax.ShapeDtypeStruct((NPF, 128), jnp.float32),
            jax.ShapeDtypeStruct((NPF, 128), jnp.float32),
        ],
    )(xf, dvf, a0, a1)


def _tc2_call(z0f, z1f, h0f, h1f, dvf, b0, b1, w00, w01, w10, w11):
    def body(z0_ref, z1_ref, h0_ref, h1_ref, dv_ref, b0_ref, b1_ref,
             w00_ref, w01_ref, w10_ref, w11_ref, g0_ref, g1_ref):
        dv = dv_ref[...]
        a0 = jax.nn.sigmoid((z0_ref[...] + h0_ref[...]) * dv + b0_ref[...])
        a1 = jax.nn.sigmoid((z1_ref[...] + h1_ref[...]) * dv + b1_ref[...])
        g0_ref[...] = (
            jnp.dot(a0, w00_ref[...], preferred_element_type=jnp.float32)
            + jnp.dot(a1, w10_ref[...], preferred_element_type=jnp.float32)
        ) * dv
        g1_ref[...] = (
            jnp.dot(a0, w01_ref[...], preferred_element_type=jnp.float32)
            + jnp.dot(a1, w11_ref[...], preferred_element_type=jnp.float32)
        ) * dv

    return pl.pallas_call(
        body,
        grid=(GRID,),
        in_specs=[
            _nspec(), _nspec(), _nspec(), _nspec(), _nspec(),
            _wspec((1, 128)), _wspec((1, 128)),
            _wspec((128, 128)), _wspec((128, 128)),
            _wspec((128, 128)), _wspec((128, 128)),
        ],
        out_specs=[_nspec(), _nspec()],
        out_shape=[
            jax.ShapeDtypeStruct((NPF, 128), jnp.float32),
            jax.ShapeDtypeStruct((NPF, 128), jnp.float32),
        ],
    )(z0f, z1f, h0f, h1f, dvf, b0, b1, w00, w01, w10, w11)


def _tc3_call(y0f, y1f, g0f, g1f, dvf, b0, b1, l0, l1, blx):
    def body(y0_ref, y1_ref, g0_ref, g1_ref, dv_ref, b0_ref, b1_ref,
             l0_ref, l1_ref, bl_ref, o_ref):
        dv = dv_ref[...]
        a0 = jax.nn.sigmoid((y0_ref[...] + g0_ref[...]) * dv + b0_ref[...])
        a1 = jax.nn.sigmoid((y1_ref[...] + g1_ref[...]) * dv + b1_ref[...])
        o_ref[...] = jax.nn.sigmoid(
            jnp.dot(a0, l0_ref[...], preferred_element_type=jnp.float32)
            + jnp.dot(a1, l1_ref[...], preferred_element_type=jnp.float32)
            + bl_ref[...]
        )

    return pl.pallas_call(
        body,
        grid=(GRID,),
        in_specs=[
            _nspec(), _nspec(), _nspec(), _nspec(), _nspec(),
            _wspec((1, 128)), _wspec((1, 128)),
            _wspec((128, 2 * 8)), _wspec((128, 2 * 8)), _wspec((1, 2 * 8)),
        ],
        out_specs=pl.BlockSpec((BLK, 2 * 8), lambda i: (i, 0)),
        out_shape=jax.ShapeDtypeStruct((NPF, 2 * 8), jnp.float32),
    )(y0f, y1f, g0f, g1f, dvf, b0, b1, l0, l1, blx)


def kernel(x, edge_index, W1, b1, W2, b2, Wl, bl):
    f32 = jnp.float32
    eiflat = edge_index.astype(jnp.int32).reshape(2 * E)

    degp = _deg_kernel(eiflat)
    dvf = _dinv_kernel(degp)

    xf = jnp.pad(x, ((0, NP - N), (0, 0))).reshape(NPF, 128)
    eye8 = jnp.eye(8, dtype=f32)
    a0 = jnp.kron(eye8, W1[:, :HALF])
    a1 = jnp.kron(eye8, W1[:, HALF:])
    h0f, h1f = _tc1_call(xf, dvf, a0, a1)

    z0, z1 = _agg_kernel(
        h0f.reshape(NP, HALF), h1f.reshape(NP, HALF), eiflat, eiflat
    )

    b1x0 = jnp.tile(b1[:HALF], 8).reshape(1, 128)
    b1x1 = jnp.tile(b1[HALF:], 8).reshape(1, 128)
    w00 = jnp.kron(eye8, W2[:HALF, :HALF])
    w01 = jnp.kron(eye8, W2[:HALF, HALF:])
    w10 = jnp.kron(eye8, W2[HALF:, :HALF])
    w11 = jnp.kron(eye8, W2[HALF:, HALF:])
    g0f, g1f = _tc2_call(
        z0.reshape(NPF, 128), z1.reshape(NPF, 128), h0f, h1f, dvf,
        b1x0, b1x1, w00, w01, w10, w11,
    )

    y0, y1 = _agg_kernel(
        g0f.reshape(NP, HALF), g1f.reshape(NP, HALF), eiflat, eiflat
    )

    b2x0 = jnp.tile(b2[:HALF], 8).reshape(1, 128)
    b2x1 = jnp.tile(b2[HALF:], 8).reshape(1, 128)
    l0 = jnp.kron(eye8, Wl[:HALF, :])
    l1 = jnp.kron(eye8, Wl[HALF:, :])
    blx = jnp.tile(bl, 8).reshape(1, 16)
    outf = _tc3_call(
        y0.reshape(NPF, 128), y1.reshape(NPF, 128), g0f, g1f, dvf,
        b2x0, b2x1, l0, l1, blx,
    )
    return outf.reshape(NP, FOUT)[:N]
